# Initial kernel scaffold; baseline (speedup 1.0000x reference)
import jax
import jax.numpy as jnp
from jax.experimental import pallas as pl


def kernel(x, attention_query, cu_seqlens, attn_weight):
    raise NotImplementedError("write your pallas kernel here")


# speedup vs baseline: 11.4342x; 11.4342x over previous
import functools

import jax
import jax.numpy as jnp
from jax import lax
from jax.experimental import pallas as pl
from jax.experimental.pallas import tpu as pltpu

N = 32768
B = 16
D = 128
GC = 512
CH = 1024
NCHUNK = N // CH
NEG = -1e30


def _body(x_ref, q_ref, cu_ref, w_ref, out_ref, m_ref, d_ref, a_ref):
    i = pl.program_id(0)

    @pl.when(i == 0)
    def _init():
        m_ref[...] = jnp.full((3, B), NEG, jnp.float32)
        d_ref[...] = jnp.zeros((3, B), jnp.float32)
        a_ref[...] = jnp.zeros((3, B, D), jnp.float32)

    x_c = x_ref[...]
    tok = i * CH + lax.broadcasted_iota(jnp.int32, (CH, 1), 0)
    lo = jnp.array([0], jnp.int32)
    cu = jnp.stack([cu_ref[b] for b in range(B + 1)])
    lo = cu[:B][None, :]
    hi = cu[1:][None, :]
    mask = (tok >= lo) & (tok < hi)
    maskf = mask.astype(jnp.float32)

    p = lax.dot_general(x_c, w_ref[...], (((1,), (1,)), ((), ())),
                        preferred_element_type=jnp.float32)
    gcol = lax.broadcasted_iota(jnp.int32, (CH, GC), 1)
    q_c = q_ref[...]

    for l in range(3):
        onehot = (gcol == q_c[:, l][:, None])
        logit = jnp.sum(jnp.where(onehot, p, 0.0), axis=1, keepdims=True)

        lmask = jnp.where(mask, logit, NEG)
        cmax = jnp.max(lmask, axis=0)
        m_old = m_ref[l, :]
        m_new = jnp.maximum(m_old, cmax)
        scale = jnp.exp(m_old - m_new)
        e = jnp.where(mask, jnp.exp(logit - m_new[None, :]), 0.0)
        d_new = d_ref[l, :] * scale + jnp.sum(e, axis=0)
        contrib = lax.dot_general(e, x_c, (((0,), (0,)), ((), ())),
                                  preferred_element_type=jnp.float32)
        a_new = a_ref[l] * scale[:, None] + contrib
        m_ref[l, :] = m_new
        d_ref[l, :] = d_new
        a_ref[l] = a_new

        @pl.when(i == NCHUNK - 1)
        def _fin():
            denom = d_new[:, None]
            out_ref[l] = jnp.where(denom > 0.0, a_new / denom, 0.0)


@jax.jit
def _run(x, attention_query, cu_seqlens, attn_weight):
    out = pl.pallas_call(
        _body,
        grid=(NCHUNK,),
        in_specs=[
            pl.BlockSpec((CH, D), lambda i: (i, 0)),
            pl.BlockSpec((CH, 3), lambda i: (i, 0)),
            pl.BlockSpec(memory_space=pltpu.SMEM),
            pl.BlockSpec((GC, D), lambda i: (0, 0)),
        ],
        out_specs=pl.BlockSpec((3, B, D), lambda i: (0, 0, 0)),
        out_shape=jax.ShapeDtypeStruct((3, B, D), jnp.float32),
        scratch_shapes=[
            pltpu.VMEM((3, B), jnp.float32),
            pltpu.VMEM((3, B), jnp.float32),
            pltpu.VMEM((3, B, D), jnp.float32),
        ],
    )(x, attention_query, cu_seqlens, attn_weight)
    return out


def kernel(x, attention_query, cu_seqlens, attn_weight):
    return (_run(x, attention_query, cu_seqlens, attn_weight), None, None)
